# baseline (speedup 1.0000x reference)
import jax
import jax.numpy as jnp
from jax import lax
from jax.experimental import pallas as pl
from jax.experimental.pallas import tpu as pltpu
from jax.experimental.pallas import tpu_sc as plsc

_B = 16384
_N = 1000000
_F = 32
_L = 16
_NC = 2
_NS = 16
_NW = _NC * _NS
_BPW = _B // _NW
_ICH = 128
_NCHUNK = _BPW // _ICH
_BROW = _L

_SCAN_BLK = 16384
_SCAN_GRID = (_N + _SCAN_BLK - 1) // _SCAN_BLK


def _dot_body(wt_ref, ht_ref, wb_ref, hb_ref, out_ref):
    out_ref[...] = (jnp.sum(wt_ref[...] * ht_ref[...], axis=0)
                    + wb_ref[...] + hb_ref[...])


_dot_scan = pl.pallas_call(
    _dot_body,
    out_shape=jax.ShapeDtypeStruct((_N,), jnp.float32),
    grid=(_SCAN_GRID,),
    in_specs=[
        pl.BlockSpec((_F, _SCAN_BLK), lambda i: (0, i)),
        pl.BlockSpec((_F, _SCAN_BLK), lambda i: (0, i)),
        pl.BlockSpec((_SCAN_BLK,), lambda i: (i,)),
        pl.BlockSpec((_SCAN_BLK,), lambda i: (i,)),
    ],
    out_specs=pl.BlockSpec((_SCAN_BLK,), lambda i: (i,)),
)

_mesh = plsc.VectorSubcoreMesh(core_axis_name="c", subcore_axis_name="s")

_SCRATCH = [
    pltpu.VMEM((_NCHUNK, _ICH), jnp.int32),
    pltpu.VMEM((_NCHUNK, _ICH), jnp.int32),
    pltpu.VMEM((_BPW, _BROW), jnp.float32),
    pltpu.VMEM((_BPW,), jnp.float32),
    pltpu.SemaphoreType.DMA,
]


def _pick_body(nodes_hbm, dt_hbm, out_hbm,
               idx_v, div_v, d_rows, out_v, sem):
    wid = lax.axis_index("s") * _NC + lax.axis_index("c")

    pltpu.sync_copy(nodes_hbm.at[pl.ds(wid * _NCHUNK, _NCHUNK)], idx_v)

    for k in range(_NCHUNK):
        for c in range(_ICH // _L):
            sl = pl.ds(c * _L, _L)
            div_v[k, sl] = lax.shift_right_logical(idx_v[k, sl], 4)

    copies = []
    for k in range(_NCHUNK):
        rows = pl.ds(k * _ICH, _ICH)
        copies.append(pltpu.async_copy(dt_hbm.at[div_v.at[k]],
                                       d_rows.at[rows], sem))
    for c in copies:
        c.wait()

    lane = lax.iota(jnp.int32, _L)
    gdn = lax.GatherDimensionNumbers(
        offset_dims=(), collapsed_slice_dims=(0,), start_index_map=(0,))

    def _permute(v, perm2d):
        return lax.gather(v, perm2d, gdn, slice_sizes=(1,),
                          mode=lax.GatherScatterMode.PROMISE_IN_BOUNDS)

    bcast = [jnp.full((_L, 1), j, jnp.int32) for j in range(_L)]
    zero = jnp.zeros((_L,), jnp.float32)

    def group_body(g, carry):
        rbase = g * _L
        nid = idx_v[g // (_ICH // _L), pl.ds((g % (_ICH // _L)) * _L, _L)]
        col = jnp.bitwise_and(nid, _BROW - 1)
        acc = zero
        for j in range(_L):
            srow = d_rows[rbase + j, :]
            colj = _permute(col, bcast[j])
            val = _permute(srow, colj[:, None])
            acc = jnp.where(lane == j, val, acc)
        out_v[pl.ds(rbase, _L)] = acc
        return carry

    lax.fori_loop(0, _BPW // _L, group_body, None)

    pltpu.sync_copy(out_v, out_hbm.at[pl.ds(wid * _BPW, _BPW)])


_pick_sc = pl.kernel(
    _pick_body,
    out_type=jax.ShapeDtypeStruct((_B,), jnp.float32),
    mesh=_mesh,
    compiler_params=pltpu.CompilerParams(use_tc_tiling_on_sc=False),
    scratch_types=_SCRATCH,
)


def kernel(nodes, W, H, w_bias, h_bias):
    dotall = _dot_scan(W.T, H.T, w_bias.reshape(-1), h_bias.reshape(-1))
    nodes2 = nodes.astype(jnp.int32).reshape(_NW * _NCHUNK, _ICH)
    return _pick_sc(nodes2, dotall.reshape(-1, _BROW))


# speedup vs baseline: 1.0409x; 1.0409x over previous
import jax
import jax.numpy as jnp
from jax import lax
from jax.experimental import pallas as pl
from jax.experimental.pallas import tpu as pltpu
from jax.experimental.pallas import tpu_sc as plsc

_B = 16384
_N = 1000000
_F = 32
_L = 16
_NC = 2
_NS = 16
_NW = _NC * _NS
_BPW = _B // _NW
_ICH = 128
_NCHUNK = _BPW // _ICH
_BROW = _L

_SCAN_BLK = 32768
_SCAN_GRID = (_N + _SCAN_BLK - 1) // _SCAN_BLK


def _dot_body(wt_ref, ht_ref, wb_ref, hb_ref, out_ref):
    out_ref[...] = (jnp.sum(wt_ref[...] * ht_ref[...], axis=0)
                    + wb_ref[...] + hb_ref[...])


_dot_scan = pl.pallas_call(
    _dot_body,
    out_shape=jax.ShapeDtypeStruct((_N,), jnp.float32),
    grid=(_SCAN_GRID,),
    in_specs=[
        pl.BlockSpec((_F, _SCAN_BLK), lambda i: (0, i)),
        pl.BlockSpec((_F, _SCAN_BLK), lambda i: (0, i)),
        pl.BlockSpec((_SCAN_BLK,), lambda i: (i,)),
        pl.BlockSpec((_SCAN_BLK,), lambda i: (i,)),
    ],
    out_specs=pl.BlockSpec((_SCAN_BLK,), lambda i: (i,)),
)

_mesh = plsc.VectorSubcoreMesh(core_axis_name="c", subcore_axis_name="s")

_SCRATCH = [
    pltpu.VMEM((_NCHUNK, _ICH), jnp.int32),
    pltpu.VMEM((_NCHUNK, _ICH), jnp.int32),
    pltpu.VMEM((_BPW, _BROW), jnp.float32),
    pltpu.VMEM((_BPW,), jnp.float32),
    pltpu.SemaphoreType.DMA,
]


def _pick_body(nodes_hbm, dt_hbm, out_hbm,
               idx_v, div_v, d_rows, out_v, sem):
    wid = lax.axis_index("s") * _NC + lax.axis_index("c")

    pltpu.sync_copy(nodes_hbm.at[pl.ds(wid * _NCHUNK, _NCHUNK)], idx_v)

    for k in range(_NCHUNK):
        for c in range(_ICH // _L):
            sl = pl.ds(c * _L, _L)
            div_v[k, sl] = lax.shift_right_logical(idx_v[k, sl], 4)

    copies = []
    for k in range(_NCHUNK):
        rows = pl.ds(k * _ICH, _ICH)
        copies.append(pltpu.async_copy(dt_hbm.at[div_v.at[k]],
                                       d_rows.at[rows], sem))
    for c in copies:
        c.wait()

    lane = lax.iota(jnp.int32, _L)
    gdn = lax.GatherDimensionNumbers(
        offset_dims=(), collapsed_slice_dims=(0,), start_index_map=(0,))

    def _permute(v, perm2d):
        return lax.gather(v, perm2d, gdn, slice_sizes=(1,),
                          mode=lax.GatherScatterMode.PROMISE_IN_BOUNDS)

    bcast = [jnp.full((_L, 1), j, jnp.int32) for j in range(_L)]
    zero = jnp.zeros((_L,), jnp.float32)

    def group_body(g, carry):
        rbase = g * _L
        nid = idx_v[g // (_ICH // _L), pl.ds((g % (_ICH // _L)) * _L, _L)]
        col = jnp.bitwise_and(nid, _BROW - 1)
        acc = zero
        for j in range(_L):
            srow = d_rows[rbase + j, :]
            colj = _permute(col, bcast[j])
            val = _permute(srow, colj[:, None])
            acc = jnp.where(lane == j, val, acc)
        out_v[pl.ds(rbase, _L)] = acc
        return carry

    lax.fori_loop(0, _BPW // _L, group_body, None)

    pltpu.sync_copy(out_v, out_hbm.at[pl.ds(wid * _BPW, _BPW)])


_pick_sc = pl.kernel(
    _pick_body,
    out_type=jax.ShapeDtypeStruct((_B,), jnp.float32),
    mesh=_mesh,
    compiler_params=pltpu.CompilerParams(use_tc_tiling_on_sc=False),
    scratch_types=_SCRATCH,
)


def kernel(nodes, W, H, w_bias, h_bias):
    dotall = _dot_scan(W.T, H.T, w_bias.reshape(-1), h_bias.reshape(-1))
    nodes2 = nodes.astype(jnp.int32).reshape(_NW * _NCHUNK, _ICH)
    return _pick_sc(nodes2, dotall.reshape(-1, _BROW))
